# baseline (speedup 1.0000x reference)
import functools

import jax
import jax.numpy as jnp
from jax import lax
from jax.experimental import pallas as pl
from jax.experimental.pallas import tpu as pltpu
from jax.experimental.pallas import tpu_sc as plsc

_NC = 2
_NS = 16
_L = 16
_UNROLL = 8


def _prep(x, ei, w1, b1, w2, b2, e_pad):
    n, d = x.shape
    grid = 5
    eblk = e_pad // grid
    nblk = 2048
    n_pad = grid * nblk
    assert grid * eblk == e_pad and n_pad >= n

    e_true = ei.shape[1]

    def mk(x_ref, ei_ref, w1_ref, b1_ref, w2_ref, b2_ref,
           m_ref, s_ref, d_ref, w1a_ref, w2a_ref):
        b = ei_ref[...]
        s_ref[...] = b[0:1, :]
        d_ref[...] = b[1:2, :]

        @pl.when(pl.program_id(0) == grid - 1)
        def _():
            col = (lax.broadcasted_iota(jnp.int32, (1, eblk), 1)
                   + (grid - 1) * eblk)
            valid = col < e_true
            s_ref[...] = jnp.where(valid, b[0:1, :], 0)
            d_ref[...] = jnp.where(valid, b[1:2, :], 0)
        ones_d = jnp.ones((1, d), jnp.float32)
        m_ref[...] = lax.dot_general(
            ones_d, x_ref[...], (((1,), (1,)), ((), ())),
            preferred_element_type=jnp.float32) * (1.0 / d)
        w1a_ref[...] = jnp.concatenate(
            [w1_ref[...], b1_ref[...].reshape(1, -1)], axis=0)
        w2a_ref[...] = jnp.concatenate(
            [w2_ref[...], b2_ref[...].reshape(1, 1)], axis=1)

    return pl.pallas_call(
        mk,
        grid=(grid,),
        in_specs=[
            pl.BlockSpec((nblk, d), lambda i: (i, 0)),
            pl.BlockSpec((2, eblk), lambda i: (0, i)),
            pl.BlockSpec(w1.shape, lambda i: (0, 0)),
            pl.BlockSpec(b1.shape, lambda i: (0,)),
            pl.BlockSpec(w2.shape, lambda i: (0, 0)),
            pl.BlockSpec(b2.shape, lambda i: (0,)),
        ],
        out_specs=(pl.BlockSpec((1, nblk), lambda i: (0, i)),
                   pl.BlockSpec((1, eblk), lambda i: (0, i)),
                   pl.BlockSpec((1, eblk), lambda i: (0, i)),
                   pl.BlockSpec((5, 32), lambda i: (0, 0)),
                   pl.BlockSpec((1, 33), lambda i: (0, 0))),
        out_shape=(jax.ShapeDtypeStruct((1, n_pad), jnp.float32),
                   jax.ShapeDtypeStruct((1, e_pad), jnp.int32),
                   jax.ShapeDtypeStruct((1, e_pad), jnp.int32),
                   jax.ShapeDtypeStruct((5, 32), jnp.float32),
                   jax.ShapeDtypeStruct((1, 33), jnp.float32)),
        compiler_params=pltpu.CompilerParams(
            dimension_semantics=("parallel",)),
    )(x, ei, w1, b1, w2, b2)


def _sc_hd_aa(m, areas, src, dst, n):
    e = src.shape[1]
    n_pad = m.shape[1]
    nw = _NC * _NS
    n_per = e // nw
    ch = 2560
    n_chunks = n_per // ch
    assert n_per * nw == e and n_chunks * ch == n_per
    assert ch % (_L * _UNROLL) == 0

    mesh = plsc.VectorSubcoreMesh(core_axis_name="c", subcore_axis_name="s")

    @functools.partial(
        pl.kernel,
        mesh=mesh,
        compiler_params=pltpu.CompilerParams(needs_layout_passes=False),
        out_type=(
            jax.ShapeDtypeStruct((e,), jnp.float32),
            jax.ShapeDtypeStruct((e,), jnp.float32),
        ),
        scratch_types=[
            pltpu.VMEM((1, n_pad), jnp.float32),
            pltpu.VMEM((n,), jnp.float32),
            pltpu.VMEM((ch,), jnp.int32),
            pltpu.VMEM((ch,), jnp.int32),
            pltpu.VMEM((ch,), jnp.int32),
            pltpu.VMEM((ch,), jnp.int32),
            pltpu.VMEM((ch,), jnp.float32),
            pltpu.VMEM((ch,), jnp.float32),
            pltpu.VMEM((ch,), jnp.float32),
            pltpu.VMEM((ch,), jnp.float32),
            pltpu.SemaphoreType.DMA,
            pltpu.SemaphoreType.DMA,
            pltpu.SemaphoreType.DMA,
            pltpu.SemaphoreType.DMA,
        ],
    )
    def k(m_hbm, a_hbm, src_hbm, dst_hbm, hd_hbm, aa_hbm,
          m_v, a_v, s0, s1, d0, d1, h0, h1, a0, a1,
          sin0, sin1, sout0, sout1):
        wid = lax.axis_index("s") * _NC + lax.axis_index("c")
        base = wid * n_per
        pltpu.sync_copy(m_hbm, m_v)
        pltpu.sync_copy(a_hbm, a_v)
        sbufs, dbufs = (s0, s1), (d0, d1)
        hbufs, abufs = (h0, h1), (a0, a1)
        sins, souts = (sin0, sin1), (sout0, sout1)

        def cb_of(c):
            return pl.multiple_of(base + c * ch, 128)

        def start_in(c):
            b = c % 2
            pltpu.async_copy(src_hbm.at[0, pl.ds(cb_of(c), ch)], sbufs[b], sins[b])
            pltpu.async_copy(dst_hbm.at[0, pl.ds(cb_of(c), ch)], dbufs[b], sins[b])

        start_in(0)
        out_handles = [None, None]
        for c in range(n_chunks):
            b = c % 2
            if c + 1 < n_chunks:
                start_in(c + 1)
            pltpu.make_async_copy(
                src_hbm.at[0, pl.ds(cb_of(c), ch)], sbufs[b], sins[b]).wait()
            pltpu.make_async_copy(
                dst_hbm.at[0, pl.ds(cb_of(c), ch)], dbufs[b], sins[b]).wait()
            if out_handles[b] is not None:
                for hnd in out_handles[b]:
                    hnd.wait()
            src_v, dst_v = sbufs[b], dbufs[b]
            hd_v, aa_v = hbufs[b], abufs[b]

            m_row = m_v.at[0]

            @plsc.parallel_loop(0, ch, step=_L, unroll=_UNROLL)
            def vec_body(o):
                s = src_v[pl.ds(o, _L)]
                d = dst_v[pl.ds(o, _L)]
                ms = plsc.load_gather(m_row, [s])
                md = plsc.load_gather(m_row, [d])
                sa = plsc.load_gather(a_v, [s])
                da = plsc.load_gather(a_v, [d])
                hd_v[pl.ds(o, _L)] = ms - md
                aa_v[pl.ds(o, _L)] = 2.0 * sa * da / (sa + da + 1e-8)

            out_handles[b] = (
                pltpu.async_copy(hd_v, hd_hbm.at[pl.ds(cb_of(c), ch)], souts[b]),
                pltpu.async_copy(aa_v, aa_hbm.at[pl.ds(cb_of(c), ch)], souts[b]),
            )
        for hs in out_handles:
            if hs is not None:
                for hnd in hs:
                    hnd.wait()

    return k(m, areas, src, dst)


def _mlp(hd, aa, ea_t, w1a, w2a):
    e = ea_t.shape[1]
    blk = 65536
    grid = -(-e // blk)

    def mk(hd_ref, aa_ref, ea_ref, w1a_ref, w2a_ref, o_ref):
        hdb = hd_ref[...].reshape(1, blk)
        aab = aa_ref[...].reshape(1, blk)
        eab = ea_ref[...]
        ones = jnp.ones((1, blk), jnp.float32)
        fb = jnp.concatenate([hdb, eab[0:1, :], eab[1:2, :], aab, ones], axis=0)
        h = lax.dot_general(w1a_ref[...], fb, (((0,), (0,)), ((), ())),
                            preferred_element_type=jnp.float32)
        h = jnp.maximum(h, 0.0)
        ha = jnp.concatenate([h, ones], axis=0)
        o_ref[...] = lax.dot_general(w2a_ref[...], ha, (((1,), (0,)), ((), ())),
                                     preferred_element_type=jnp.float32)

    return pl.pallas_call(
        mk,
        grid=(grid,),
        in_specs=[
            pl.BlockSpec((blk,), lambda i: (i,)),
            pl.BlockSpec((blk,), lambda i: (i,)),
            pl.BlockSpec((4, blk), lambda i: (0, i)),
            pl.BlockSpec((5, 32), lambda i: (0, 0)),
            pl.BlockSpec((1, 33), lambda i: (0, 0)),
        ],
        out_specs=pl.BlockSpec((1, blk), lambda i: (0, i)),
        out_shape=jax.ShapeDtypeStruct((1, e), jnp.float32),
        compiler_params=pltpu.CompilerParams(
            dimension_semantics=("parallel",)),
    )(hd, aa, ea_t, w1a, w2a)


def kernel(x, edge_index, edge_attr, node_areas, W1, b1, W2, b2):
    e = edge_index.shape[1]
    n = x.shape[0]
    e_pad = -(-e // 65536) * 65536
    m, src, dst, w1a, w2a = _prep(
        x.astype(jnp.float32), edge_index.astype(jnp.int32),
        W1.astype(jnp.float32), b1.astype(jnp.float32),
        W2.astype(jnp.float32).reshape(1, -1),
        b2.astype(jnp.float32), e_pad)
    hd, aa = _sc_hd_aa(m, node_areas.astype(jnp.float32), src, dst, n)
    out = _mlp(hd, aa, edge_attr.astype(jnp.float32).T, w1a, w2a)
    return out.reshape(e, 1)


# speedup vs baseline: 1.0128x; 1.0128x over previous
import functools

import jax
import jax.numpy as jnp
from jax import lax
from jax.experimental import pallas as pl
from jax.experimental.pallas import tpu as pltpu
from jax.experimental.pallas import tpu_sc as plsc

_NC = 2
_NS = 16
_L = 16
_UNROLL = 8


def _prep(x, ei, w1, b1, w2, b2, e_pad):
    n, d = x.shape
    grid = 5
    eblk = e_pad // grid
    nblk = 2048
    n_pad = grid * nblk
    assert grid * eblk == e_pad and n_pad >= n

    def mk(x_ref, ei_ref, w1_ref, b1_ref, w2_ref, b2_ref,
           m_ref, s_ref, d_ref, w1a_ref, w2a_ref):
        b = ei_ref[...]
        s_ref[...] = b[0:1, :]
        d_ref[...] = b[1:2, :]
        ones_d = jnp.ones((1, d), jnp.float32)
        m_ref[...] = lax.dot_general(
            ones_d, x_ref[...], (((1,), (1,)), ((), ())),
            preferred_element_type=jnp.float32) * (1.0 / d)
        w1a_ref[...] = jnp.concatenate(
            [w1_ref[...], b1_ref[...].reshape(1, -1)], axis=0)
        w2a_ref[...] = jnp.concatenate(
            [w2_ref[...], b2_ref[...].reshape(1, 1)], axis=1)

    return pl.pallas_call(
        mk,
        grid=(grid,),
        in_specs=[
            pl.BlockSpec((nblk, d), lambda i: (i, 0)),
            pl.BlockSpec((2, eblk), lambda i: (0, i)),
            pl.BlockSpec(w1.shape, lambda i: (0, 0)),
            pl.BlockSpec(b1.shape, lambda i: (0,)),
            pl.BlockSpec(w2.shape, lambda i: (0, 0)),
            pl.BlockSpec(b2.shape, lambda i: (0,)),
        ],
        out_specs=(pl.BlockSpec((1, nblk), lambda i: (0, i)),
                   pl.BlockSpec((1, eblk), lambda i: (0, i)),
                   pl.BlockSpec((1, eblk), lambda i: (0, i)),
                   pl.BlockSpec((5, 32), lambda i: (0, 0)),
                   pl.BlockSpec((1, 33), lambda i: (0, 0))),
        out_shape=(jax.ShapeDtypeStruct((1, n_pad), jnp.float32),
                   jax.ShapeDtypeStruct((1, e_pad), jnp.int32),
                   jax.ShapeDtypeStruct((1, e_pad), jnp.int32),
                   jax.ShapeDtypeStruct((5, 32), jnp.float32),
                   jax.ShapeDtypeStruct((1, 33), jnp.float32)),
        compiler_params=pltpu.CompilerParams(
            dimension_semantics=("parallel",)),
    )(x, ei, w1, b1, w2, b2)


def _sc_hd_aa(m, areas, src, dst, n):
    e = src.shape[1]
    n_pad = m.shape[1]
    nw = _NC * _NS
    n_per = e // nw
    ch = 2560
    n_chunks = n_per // ch
    assert n_per * nw == e and n_chunks * ch == n_per
    assert ch % (_L * _UNROLL) == 0

    mesh = plsc.VectorSubcoreMesh(core_axis_name="c", subcore_axis_name="s")

    @functools.partial(
        pl.kernel,
        mesh=mesh,
        compiler_params=pltpu.CompilerParams(needs_layout_passes=False),
        out_type=(
            jax.ShapeDtypeStruct((e,), jnp.float32),
            jax.ShapeDtypeStruct((e,), jnp.float32),
        ),
        scratch_types=[
            pltpu.VMEM((1, n_pad), jnp.float32),
            pltpu.VMEM((n,), jnp.float32),
            pltpu.VMEM((ch,), jnp.int32),
            pltpu.VMEM((ch,), jnp.int32),
            pltpu.VMEM((ch,), jnp.int32),
            pltpu.VMEM((ch,), jnp.int32),
            pltpu.VMEM((ch,), jnp.float32),
            pltpu.VMEM((ch,), jnp.float32),
            pltpu.VMEM((ch,), jnp.float32),
            pltpu.VMEM((ch,), jnp.float32),
            pltpu.SemaphoreType.DMA,
            pltpu.SemaphoreType.DMA,
            pltpu.SemaphoreType.DMA,
            pltpu.SemaphoreType.DMA,
        ],
    )
    def k(m_hbm, a_hbm, src_hbm, dst_hbm, hd_hbm, aa_hbm,
          m_v, a_v, s0, s1, d0, d1, h0, h1, a0, a1,
          sin0, sin1, sout0, sout1):
        wid = lax.axis_index("s") * _NC + lax.axis_index("c")
        base = wid * n_per
        pltpu.sync_copy(m_hbm, m_v)
        pltpu.sync_copy(a_hbm, a_v)
        sbufs, dbufs = (s0, s1), (d0, d1)
        hbufs, abufs = (h0, h1), (a0, a1)
        sins, souts = (sin0, sin1), (sout0, sout1)

        def cb_of(c):
            return pl.multiple_of(base + c * ch, 128)

        def start_in(c):
            b = c % 2
            pltpu.async_copy(src_hbm.at[0, pl.ds(cb_of(c), ch)], sbufs[b], sins[b])
            pltpu.async_copy(dst_hbm.at[0, pl.ds(cb_of(c), ch)], dbufs[b], sins[b])

        start_in(0)
        out_handles = [None, None]
        for c in range(n_chunks):
            b = c % 2
            if c + 1 < n_chunks:
                start_in(c + 1)
            pltpu.make_async_copy(
                src_hbm.at[0, pl.ds(cb_of(c), ch)], sbufs[b], sins[b]).wait()
            pltpu.make_async_copy(
                dst_hbm.at[0, pl.ds(cb_of(c), ch)], dbufs[b], sins[b]).wait()
            if out_handles[b] is not None:
                for hnd in out_handles[b]:
                    hnd.wait()
            src_v, dst_v = sbufs[b], dbufs[b]
            hd_v, aa_v = hbufs[b], abufs[b]

            zero = jnp.zeros((_L,), jnp.int32)

            @plsc.parallel_loop(0, ch, step=_L, unroll=_UNROLL)
            def vec_body(o):
                s = jnp.minimum(jnp.maximum(src_v[pl.ds(o, _L)], 0), n - 1)
                d = jnp.minimum(jnp.maximum(dst_v[pl.ds(o, _L)], 0), n - 1)
                ms = plsc.load_gather(m_v, [zero, s])
                md = plsc.load_gather(m_v, [zero, d])
                sa = plsc.load_gather(a_v, [s])
                da = plsc.load_gather(a_v, [d])
                hd_v[pl.ds(o, _L)] = ms - md
                aa_v[pl.ds(o, _L)] = 2.0 * sa * da / (sa + da + 1e-8)

            out_handles[b] = (
                pltpu.async_copy(hd_v, hd_hbm.at[pl.ds(cb_of(c), ch)], souts[b]),
                pltpu.async_copy(aa_v, aa_hbm.at[pl.ds(cb_of(c), ch)], souts[b]),
            )
        for hs in out_handles:
            if hs is not None:
                for hnd in hs:
                    hnd.wait()

    return k(m, areas, src, dst)


def _mlp(hd, aa, ea_t, w1a, w2a):
    e = ea_t.shape[1]
    blk = 65536
    grid = -(-e // blk)

    def mk(hd_ref, aa_ref, ea_ref, w1a_ref, w2a_ref, o_ref):
        hdb = hd_ref[...].reshape(1, blk)
        aab = aa_ref[...].reshape(1, blk)
        eab = ea_ref[...]
        ones = jnp.ones((1, blk), jnp.float32)
        fb = jnp.concatenate([hdb, eab[0:1, :], eab[1:2, :], aab, ones], axis=0)
        h = lax.dot_general(w1a_ref[...], fb, (((0,), (0,)), ((), ())),
                            preferred_element_type=jnp.float32)
        h = jnp.maximum(h, 0.0)
        ha = jnp.concatenate([h, ones], axis=0)
        o_ref[...] = lax.dot_general(w2a_ref[...], ha, (((1,), (0,)), ((), ())),
                                     preferred_element_type=jnp.float32)

    return pl.pallas_call(
        mk,
        grid=(grid,),
        in_specs=[
            pl.BlockSpec((blk,), lambda i: (i,)),
            pl.BlockSpec((blk,), lambda i: (i,)),
            pl.BlockSpec((4, blk), lambda i: (0, i)),
            pl.BlockSpec((5, 32), lambda i: (0, 0)),
            pl.BlockSpec((1, 33), lambda i: (0, 0)),
        ],
        out_specs=pl.BlockSpec((1, blk), lambda i: (0, i)),
        out_shape=jax.ShapeDtypeStruct((1, e), jnp.float32),
        compiler_params=pltpu.CompilerParams(
            dimension_semantics=("parallel",)),
    )(hd, aa, ea_t, w1a, w2a)


def kernel(x, edge_index, edge_attr, node_areas, W1, b1, W2, b2):
    e = edge_index.shape[1]
    n = x.shape[0]
    e_pad = -(-e // 65536) * 65536
    m, src, dst, w1a, w2a = _prep(
        x.astype(jnp.float32), edge_index.astype(jnp.int32),
        W1.astype(jnp.float32), b1.astype(jnp.float32),
        W2.astype(jnp.float32).reshape(1, -1),
        b2.astype(jnp.float32), e_pad)
    hd, aa = _sc_hd_aa(m, node_areas.astype(jnp.float32), src, dst, n)
    out = _mlp(hd, aa, edge_attr.astype(jnp.float32).T, w1a, w2a)
    return out.reshape(e, 1)
